# baseline (speedup 1.0000x reference)
import jax
import jax.numpy as jnp
from jax import lax
from jax.experimental import pallas as pl
from jax.experimental.pallas import tpu as pltpu
from jax.experimental.pallas import tpu_sc as plsc

NC = 2
NS = 16
NT = NC * NS
K = 128
NSPLIT = 2
EPS = 1e-5


def _mesh():
  return plsc.VectorSubcoreMesh(core_axis_name="c", subcore_axis_name="s",
                                num_cores=NC, num_subcores=NS)


def _make_sc_agg(n, d, n_pad, iters):
  rpt = n_pad // NS
  assert iters % 4 == 0
  ipp = iters // 2
  scratch = [
      pltpu.VMEM((ipp, K), jnp.int32),
      pltpu.VMEM((ipp, K), jnp.int32),
      pltpu.VMEM((K, d), jnp.float32),
      pltpu.VMEM((K, d), jnp.float32),
      pltpu.VMEM_SHARED((n_pad, d), jnp.float32),
      pltpu.SemaphoreType.DMA,
      pltpu.SemaphoreType.DMA,
  ]

  def body(feat_h, src_h, dst_h, zrow_h, acc_o,
           srcv, dstv, rows_a, rows_b, accs, sem_a, sem_b):
    cid = lax.axis_index("c")
    sid = lax.axis_index("s")
    wid = cid * NS + sid
    r0 = sid * rpt
    pltpu.sync_copy(zrow_h, accs.at[pl.ds(r0, rpt)])
    plsc.subcore_barrier()

    nsteps = ipp // 2
    ksub = K // NSPLIT

    def gather(i, buf, sem):
      for u in range(NSPLIT):
        pltpu.async_copy(feat_h.at[srcv.at[i, pl.ds(u * ksub, ksub)]],
                         buf.at[pl.ds(u * ksub, ksub)], sem)

    def drain(i, buf, sem):
      for u in range(NSPLIT):
        pltpu.make_async_copy(feat_h.at[srcv.at[i, pl.ds(u * ksub, ksub)]],
                              buf.at[pl.ds(u * ksub, ksub)], sem).wait()

    for ph in range(2):
      pltpu.sync_copy(src_h.at[wid, pl.ds(ph * ipp, ipp)], srcv)
      pltpu.sync_copy(dst_h.at[wid, pl.ds(ph * ipp, ipp)], dstv)
      gather(0, rows_a, sem_a)

      def step(j, carry):
        i = 2 * j
        gather(i + 1, rows_b, sem_b)
        drain(i, rows_a, sem_a)
        pltpu.sync_copy(rows_a, accs.at[dstv.at[i]], add=True)

        @pl.when(j < nsteps - 1)
        def _():
          gather(i + 2, rows_a, sem_a)

        drain(i + 1, rows_b, sem_b)
        pltpu.sync_copy(rows_b, accs.at[dstv.at[i + 1]], add=True)
        return carry

      lax.fori_loop(0, nsteps, step, 0)
    plsc.subcore_barrier()
    pltpu.sync_copy(accs.at[pl.ds(r0, rpt)], acc_o.at[cid, pl.ds(r0, rpt)])

  return pl.kernel(
      body, out_type=jax.ShapeDtypeStruct((NC, n_pad, d), jnp.float32),
      mesh=_mesh(), scratch_types=scratch)


def _make_sc_deg(n_pad, iters, dw):
  rpt = n_pad // NS
  scratch = [
      pltpu.VMEM((iters, K), jnp.int32),
      pltpu.VMEM((K, dw), jnp.float32),
      pltpu.VMEM_SHARED((n_pad, dw), jnp.float32),
  ]

  def body(dst_h, zdeg_h, ones_h, deg_o, dstv, onesv, degs):
    cid = lax.axis_index("c")
    sid = lax.axis_index("s")
    wid = cid * NS + sid
    r0 = sid * rpt
    pltpu.sync_copy(zdeg_h, degs.at[pl.ds(r0, rpt)])
    pltpu.sync_copy(ones_h, onesv)
    pltpu.sync_copy(dst_h.at[wid], dstv)
    plsc.subcore_barrier()

    def step(i, carry):
      pltpu.sync_copy(onesv, degs.at[dstv.at[i]], add=True)
      return carry

    lax.fori_loop(0, iters, step, 0)
    plsc.subcore_barrier()
    pltpu.sync_copy(degs.at[pl.ds(r0, rpt)], deg_o.at[cid, pl.ds(r0, rpt)])

  return pl.kernel(
      body, out_type=jax.ShapeDtypeStruct((NC, n_pad, dw), jnp.float32),
      mesh=_mesh(), scratch_types=scratch)


def _tc_layer1(acc, deg, x, wlt, wrt, params, bn):
  n, d = x.shape
  grid = (n // bn,)

  def body(acc_ref, deg_ref, x_ref, wl_ref, wr_ref, p_ref, h_ref):
    p = p_ref[...]
    b, g, be, rm, rv = p[0], p[1], p[2], p[3], p[4]
    s = g * lax.rsqrt(rv + EPS)
    degs = deg_ref[0, :, 0] + deg_ref[1, :, 0]
    agg = (acc_ref[0] + acc_ref[1]) / jnp.maximum(degs, 1.0)[:, None]
    y = (jnp.dot(agg, wl_ref[...], preferred_element_type=jnp.float32)
         + jnp.dot(x_ref[...], wr_ref[...], preferred_element_type=jnp.float32))
    h_ref[...] = jnp.maximum((y + b - rm) * s + be, 0.0)

  return pl.pallas_call(
      body,
      grid=grid,
      in_specs=[
          pl.BlockSpec((NC, bn, d), lambda i: (0, i, 0)),
          pl.BlockSpec((NC, bn, d), lambda i: (0, i, 0)),
          pl.BlockSpec((bn, d), lambda i: (i, 0)),
          pl.BlockSpec((d, d), lambda i: (0, 0)),
          pl.BlockSpec((d, d), lambda i: (0, 0)),
          pl.BlockSpec((5, d), lambda i: (0, 0)),
      ],
      out_specs=pl.BlockSpec((bn, d), lambda i: (i, 0)),
      out_shape=jax.ShapeDtypeStruct((n, d), jnp.float32),
  )(acc, deg, x, wlt, wrt, params)


def _tc_layer2(acc, deg, h, wlt, wrt, params, bn):
  n, d = h.shape
  grid = (n // bn,)

  def body(acc_ref, deg_ref, h_ref, wl_ref, wr_ref, p_ref, o_ref):
    p = p_ref[...]
    b, g, be, rm, rv = p[0], p[1], p[2], p[3], p[4]
    wc_row, bc_row = p[5], p[6]
    s = g * lax.rsqrt(rv + EPS)
    degs = deg_ref[0, :, 0] + deg_ref[1, :, 0]
    agg = (acc_ref[0] + acc_ref[1]) / jnp.maximum(degs, 1.0)[:, None]
    h = h_ref[...]
    y = (jnp.dot(agg, wl_ref[...], preferred_element_type=jnp.float32)
         + jnp.dot(h, wr_ref[...], preferred_element_type=jnp.float32))
    hh = jnp.maximum((y + b - rm) * s + be + h, 0.0)
    o_ref[...] = jnp.sum(hh * wc_row, axis=1, keepdims=True) + bc_row[0]

  return pl.pallas_call(
      body,
      grid=grid,
      in_specs=[
          pl.BlockSpec((NC, bn, d), lambda i: (0, i, 0)),
          pl.BlockSpec((NC, bn, d), lambda i: (0, i, 0)),
          pl.BlockSpec((bn, d), lambda i: (i, 0)),
          pl.BlockSpec((d, d), lambda i: (0, 0)),
          pl.BlockSpec((d, d), lambda i: (0, 0)),
          pl.BlockSpec((7, d), lambda i: (0, 0)),
      ],
      out_specs=pl.BlockSpec((bn, 1), lambda i: (i, 0)),
      out_shape=jax.ShapeDtypeStruct((n, 1), jnp.float32),
  )(acc, deg, h, wlt, wrt, params)


def kernel(x, edge_index, W1l, W1r, b1, g1, be1, rm1, rv1,
           W2l, W2r, b2, g2, be2, rm2, rv2, Wc, bc):
  n, d = x.shape
  e = edge_index.shape[1]
  n_pad = ((n + 127) // 128) * 128
  iters = (e + NT * K - 1) // (NT * K)
  iters = ((iters + 3) // 4) * 4
  e_pad = iters * NT * K
  rpt = n_pad // NS

  src = edge_index[0]
  dst = edge_index[1]
  pad = e_pad - e
  if pad:
    r = jnp.arange(pad, dtype=jnp.int32)
    src = jnp.concatenate([src, r % n])
    dst = jnp.concatenate([dst, n + r % (n_pad - n)])
  src3 = src.reshape(NT, iters, K)
  dst3 = dst.reshape(NT, iters, K)

  zrow = jnp.zeros((rpt, d), jnp.float32)
  ones = jnp.ones((K, d), jnp.float32)

  sc_agg = _make_sc_agg(n, d, n_pad, iters)
  sc_deg = _make_sc_deg(n_pad, iters, d)

  deg = sc_deg(dst3, zrow, ones)
  acc1 = sc_agg(x, src3, dst3, zrow)
  p1 = jnp.stack([b1, g1, be1, rm1, rv1])
  h = _tc_layer1(acc1, deg, x, W1l.T, W1r.T, p1, bn=1000)

  acc2 = sc_agg(h, src3, dst3, zrow)
  bc_row = jnp.full((d,), bc[0], jnp.float32)
  p2 = jnp.stack([b2, g2, be2, rm2, rv2, Wc[0], bc_row])
  out2d = _tc_layer2(acc2, deg, h, W2l.T, W2r.T, p2, bn=1000)
  return out2d[:, 0]
